# baseline (speedup 1.0000x reference)
import functools

import jax
import jax.numpy as jnp
from jax import lax
from jax.experimental import pallas as pl
from jax.experimental.pallas import tpu as pltpu
from jax.experimental.pallas import tpu_sc as plsc

B = 4096 * 200
D = 32
NW = 32
B_PER_W = B // NW
ROWS_PER_W = 4096 // NW
NBUF = 4


def _make_gather_kernel():
    mesh = plsc.VectorSubcoreMesh(core_axis_name="c", subcore_axis_name="s")

    @functools.partial(
        pl.kernel,
        mesh=mesh,
        compiler_params=pltpu.CompilerParams(use_tc_tiling_on_sc=False),
        out_type=jax.ShapeDtypeStruct((4096, 200, D), jnp.float32),
        scratch_types=(
            [pltpu.VMEM((B_PER_W,), jnp.int32)]
            + [pltpu.VMEM((200, D), jnp.float32) for _ in range(NBUF)]
            + [pltpu.SemaphoreType.DMA, pltpu.SemaphoreType.DMA]
        ),
    )
    def gather_kernel(idx_hbm, table_hbm, out_hbm, idx_all, *rest):
        row_bufs = rest[:NBUF]
        gsem, osem = rest[NBUF], rest[NBUF + 1]
        wid = lax.axis_index("s") * 2 + lax.axis_index("c")
        base = wid * B_PER_W
        brow0 = wid * ROWS_PER_W
        pltpu.sync_copy(idx_hbm.at[pl.ds(base, B_PER_W)], idx_all)

        def gather(i):
            rv = row_bufs[i % NBUF]
            pltpu.async_copy(
                table_hbm.at[idx_all.at[pl.ds(i * 200, 200)]], rv, gsem)

        def gwait(i):
            rv = row_bufs[i % NBUF]
            pltpu.make_async_copy(
                table_hbm.at[idx_all.at[pl.ds(i * 200, 200)]], rv,
                gsem).wait()

        def wb_start(i):
            rv = row_bufs[i % NBUF]
            pltpu.make_async_copy(rv, out_hbm.at[brow0 + i], osem).start()

        def wb_wait(i):
            rv = row_bufs[i % NBUF]
            pltpu.make_async_copy(rv, out_hbm.at[brow0 + i], osem).wait()

        for i in range(NBUF):
            gather(i)
        for i in range(ROWS_PER_W):
            gwait(i)
            wb_start(i)
            if i + NBUF < ROWS_PER_W:
                wb_wait(i)
                gather(i + NBUF)
        for i in range(ROWS_PER_W - NBUF, ROWS_PER_W):
            wb_wait(i)

    return gather_kernel


_gather = _make_gather_kernel()


def kernel(input_x, weight):
    idx = input_x.reshape(-1).astype(jnp.int32)
    return _gather(idx, weight)
